# Initial kernel scaffold; baseline (speedup 1.0000x reference)
import jax
import jax.numpy as jnp
from jax.experimental import pallas as pl


def kernel(z, embedding):
    raise NotImplementedError("write your pallas kernel here")


# speedup vs baseline: 1.3859x; 1.3859x over previous
import jax
import jax.numpy as jnp
from jax.experimental import pallas as pl

N_E = 1024
E_DIM = 64
BETA = 0.25
BLOCK = 1024


def _vq_kernel(z_ref, emb_ref, zq_ref, idx_ref, loss_ref):
    i = pl.program_id(0)
    zb = z_ref[...]
    emb = emb_ref[...]

    z_sq = jnp.sum(zb * zb, axis=1, keepdims=True)
    e_sq = jnp.sum(emb * emb, axis=1)
    prod = jax.lax.dot_general(
        zb, emb, (((1,), (1,)), ((), ())),
        preferred_element_type=jnp.float32)
    d = z_sq + e_sq - 2.0 * prod

    idx = jnp.argmin(d, axis=1).astype(jnp.int32)
    idx_ref[0, 0, :] = idx

    cols = jax.lax.broadcasted_iota(jnp.int32, (BLOCK, N_E), 1)
    onehot = (cols == idx[:, None]).astype(jnp.float32)
    zq = jax.lax.dot_general(
        onehot, emb, (((1,), (0,)), ((), ())),
        preferred_element_type=jnp.float32)
    zq_ref[...] = zq

    diff = zq - zb
    partial = jnp.sum(diff * diff, keepdims=True).reshape(1, 1)

    @pl.when(i == 0)
    def _():
        loss_ref[...] = jnp.zeros((1, 1), jnp.float32)

    loss_ref[...] += partial


def kernel(z, embedding):
    z_flat = jnp.reshape(z, (-1, E_DIM))
    n = z_flat.shape[0]
    num_blocks = n // BLOCK

    zq_flat, idx3, loss_sum = pl.pallas_call(
        _vq_kernel,
        grid=(num_blocks,),
        in_specs=[
            pl.BlockSpec((BLOCK, E_DIM), lambda i: (i, 0)),
            pl.BlockSpec((N_E, E_DIM), lambda i: (0, 0)),
        ],
        out_specs=[
            pl.BlockSpec((BLOCK, E_DIM), lambda i: (i, 0)),
            pl.BlockSpec((1, 1, BLOCK), lambda i: (i, 0, 0)),
            pl.BlockSpec((1, 1), lambda i: (0, 0)),
        ],
        out_shape=[
            jax.ShapeDtypeStruct((n, E_DIM), jnp.float32),
            jax.ShapeDtypeStruct((num_blocks, 1, BLOCK), jnp.int32),
            jax.ShapeDtypeStruct((1, 1), jnp.float32),
        ],
    )(z_flat, embedding)

    z_q = jnp.reshape(zq_flat, z.shape)
    min_encoding_indices = jnp.reshape(idx3, (n,))
    loss = loss_sum[0, 0] * ((1.0 + BETA) / (n * E_DIM))
    return (z_q, loss, min_encoding_indices)
